# baseline (speedup 1.0000x reference)
import functools

import jax
import jax.numpy as jnp
from jax import lax
from jax.experimental import pallas as pl
from jax.experimental.pallas import tpu as pltpu
from jax.experimental.pallas import tpu_sc as plsc

NUM_LOCATION = 100000
EMBED_DIM = 128
BATCH = 16384
N_FEA = 26

NC = 2
NS = 16
NW = NC * NS
B_PER_W = BATCH // NW
NCHUNK = 4
CH = B_PER_W // NCHUNK


def _make_gather():
  mesh = plsc.VectorSubcoreMesh(core_axis_name="c", subcore_axis_name="s")

  @functools.partial(
      pl.kernel,
      out_type=jax.ShapeDtypeStruct((BATCH, EMBED_DIM), jnp.float32),
      mesh=mesh,
      scratch_types=[
          pltpu.VMEM((B_PER_W,), jnp.int32),
          pltpu.VMEM((2, CH, EMBED_DIM), jnp.float32),
          pltpu.SemaphoreType.DMA,
          pltpu.SemaphoreType.DMA,
          pltpu.SemaphoreType.DMA,
      ],
  )
  def gather_kernel(idx_hbm, table_hbm, out_hbm, idx_v, rows_v, gsem0, gsem1,
                    wsem):
    wid = lax.axis_index("s") * NC + lax.axis_index("c")
    base = wid * B_PER_W
    pltpu.sync_copy(idx_hbm.at[pl.ds(base, B_PER_W)], idx_v)

    gsems = (gsem0, gsem1)
    gathers = []
    for c in range(NCHUNK):
      gathers.append(
          pltpu.make_async_copy(
              table_hbm.at[idx_v.at[pl.ds(c * CH, CH)]],
              rows_v.at[c % 2],
              gsems[c % 2],
          )
      )
    writes = []
    for c in range(NCHUNK):
      writes.append(
          pltpu.make_async_copy(
              rows_v.at[c % 2],
              out_hbm.at[pl.ds(base + c * CH, CH)],
              wsem,
          )
      )

    gathers[0].start()
    gathers[1].start()
    for c in range(NCHUNK):
      gathers[c].wait()
      writes[c].start()
      if c + 2 < NCHUNK:
        writes[c].wait()
        gathers[c + 2].start()
    writes[NCHUNK - 2].wait()
    writes[NCHUNK - 1].wait()

  return gather_kernel


_gather = _make_gather()


@jax.jit
def kernel(user_fea, embedding_location):
  loc_idx = user_fea[:, 0].astype(jnp.int32)
  return _gather(loc_idx, embedding_location)


# speedup vs baseline: 1.0243x; 1.0243x over previous
import functools

import jax
import jax.numpy as jnp
from jax import lax
from jax.experimental import pallas as pl
from jax.experimental.pallas import tpu as pltpu
from jax.experimental.pallas import tpu_sc as plsc

NUM_LOCATION = 100000
EMBED_DIM = 128
BATCH = 16384
N_FEA = 26

NC = 2
NS = 16
NW = NC * NS
B_PER_W = BATCH // NW
NCHUNK = 4
CH = B_PER_W // NCHUNK


def _make_gather():
  mesh = plsc.VectorSubcoreMesh(core_axis_name="c", subcore_axis_name="s")

  @functools.partial(
      pl.kernel,
      out_type=jax.ShapeDtypeStruct((BATCH, EMBED_DIM), jnp.float32),
      mesh=mesh,
      scratch_types=[
          pltpu.VMEM((B_PER_W,), jnp.int32),
          pltpu.VMEM((NCHUNK, CH, EMBED_DIM), jnp.float32),
          pltpu.SemaphoreType.DMA,
          pltpu.SemaphoreType.DMA,
      ],
  )
  def gather_kernel(idx_hbm, table_hbm, out_hbm, idx_v, rows_v, gsem, wsem):
    wid = lax.axis_index("s") * NC + lax.axis_index("c")
    base = wid * B_PER_W
    pltpu.sync_copy(idx_hbm.at[pl.ds(base, B_PER_W)], idx_v)

    gathers = [
        pltpu.make_async_copy(
            table_hbm.at[idx_v.at[pl.ds(c * CH, CH)]],
            rows_v.at[c],
            gsem,
        )
        for c in range(NCHUNK)
    ]
    writes = [
        pltpu.make_async_copy(
            rows_v.at[c],
            out_hbm.at[pl.ds(base + c * CH, CH)],
            wsem,
        )
        for c in range(NCHUNK)
    ]
    for c in range(NCHUNK):
      gathers[c].start()
    for c in range(NCHUNK):
      gathers[c].wait()
      writes[c].start()
    for c in range(NCHUNK):
      writes[c].wait()

  return gather_kernel


_gather = _make_gather()


@jax.jit
def kernel(user_fea, embedding_location):
  loc_idx = user_fea[:, 0].astype(jnp.int32)
  return _gather(loc_idx, embedding_location)
